# baseline (speedup 1.0000x reference)
import functools

import jax
import jax.numpy as jnp
from jax import lax
from jax.experimental import pallas as pl
from jax.experimental.pallas import tpu as pltpu
from jax.experimental.pallas import tpu_sc as plsc

_NW = 32
_RC = 64


def _sc_body(vals_hbm, binsp_hbm, b0_hbm, bm_hbm, zeros_hbm, out_hbm,
             vals_v, bins_v, b0_v, bm_v,
             rowbuf0, rowbuf1, colbuf0, colbuf1, sem0, sem1,
             *, rows_per_w, r0, nbins):
    nchunks = rows_per_w // _RC
    wid = lax.axis_index("s") * 2 + lax.axis_index("c")
    base = wid * rows_per_w

    pltpu.sync_copy(vals_hbm.at[pl.ds(base, rows_per_w)], vals_v)
    pltpu.sync_copy(binsp_hbm, bins_v)
    pltpu.sync_copy(b0_hbm, b0_v)
    pltpu.sync_copy(bm_hbm, bm_v)
    pltpu.sync_copy(zeros_hbm, rowbuf0)
    pltpu.sync_copy(zeros_hbm, rowbuf1)

    zf = jnp.zeros((16,), jnp.float32)
    zi = jnp.zeros((16,), jnp.int32)
    for t in range(_RC // 16):
        colbuf0[pl.ds(t * 16, 16)] = zi
        colbuf1[pl.ds(t * 16, 16)] = zi

    b0 = b0_v[pl.ds(0, 16)]
    bm = bm_v[pl.ds(0, 16)]
    lane = lax.iota(jnp.int32, 16)

    def do_chunk(c, rbuf, cbuf, sem):
        @pl.when(c >= 2)
        def _wait_prior():
            pltpu.make_async_copy(
                rbuf, out_hbm.at[0, pl.ds(0, _RC)], sem).wait()

        for t in range(_RC // 16):
            rr = t * 16 + lane
            stale = cbuf[pl.ds(t * 16, 16)]
            plsc.store_scatter(rbuf, [rr, stale], zf)
            plsc.store_scatter(rbuf, [rr, stale + 1], zf)

            v = vals_v[pl.ds(c * _RC + t * 16, 16)]
            v = jnp.minimum(jnp.maximum(v, b0), bm)
            idx = zi
            for step in (128, 64, 32, 16, 8, 4, 2, 1):
                cand = idx + step
                g = plsc.load_gather(bins_v, [cand - 1])
                idx = jnp.where(g < v, cand, idx)
            left = jnp.maximum(idx - 1, 0)
            lv = plsc.load_gather(bins_v, [left])
            rv = plsc.load_gather(bins_v, [left + 1])
            rw = (v - lv) / (rv - lv)
            lw = 1.0 - rw
            plsc.store_scatter(rbuf, [rr, left], lw)
            plsc.store_scatter(rbuf, [rr, left + 1], rw)
            cbuf[pl.ds(t * 16, 16)] = left

        g0 = base + c * _RC
        p = g0 // r0
        q = g0 - p * r0
        pltpu.make_async_copy(
            rbuf, out_hbm.at[p, pl.ds(q, _RC)], sem).start()

    def chunk_pair(pp, _):
        do_chunk(2 * pp, rowbuf0, colbuf0, sem0)
        do_chunk(2 * pp + 1, rowbuf1, colbuf1, sem1)
        return 0

    lax.fori_loop(0, nchunks // 2, chunk_pair, 0)
    pltpu.make_async_copy(rowbuf0, out_hbm.at[0, pl.ds(0, _RC)], sem0).wait()
    pltpu.make_async_copy(rowbuf1, out_hbm.at[0, pl.ds(0, _RC)], sem1).wait()


def kernel(values, bin_values):
    r0, r1 = values.shape
    nbins = bin_values.shape[0]
    n = r0 * r1
    rows_per_w = n // _NW
    vt_flat = values.T.reshape(n)
    binsp = jnp.concatenate([bin_values, bin_values[-1:]])
    b0 = jnp.broadcast_to(bin_values[0], (16,))
    bm = jnp.broadcast_to(bin_values[-1], (16,))
    zeros = jnp.zeros((_RC, nbins), jnp.float32)

    mesh = plsc.VectorSubcoreMesh(core_axis_name="c", subcore_axis_name="s")
    run = pl.kernel(
        functools.partial(_sc_body, rows_per_w=rows_per_w, r0=r0,
                          nbins=nbins),
        out_type=jax.ShapeDtypeStruct((r1, r0, nbins), jnp.float32),
        mesh=mesh,
        scratch_types=[
            pltpu.VMEM((rows_per_w,), jnp.float32),
            pltpu.VMEM((nbins + 1,), jnp.float32),
            pltpu.VMEM((16,), jnp.float32),
            pltpu.VMEM((16,), jnp.float32),
            pltpu.VMEM((_RC, nbins), jnp.float32),
            pltpu.VMEM((_RC, nbins), jnp.float32),
            pltpu.VMEM((_RC,), jnp.int32),
            pltpu.VMEM((_RC,), jnp.int32),
            pltpu.SemaphoreType.DMA,
            pltpu.SemaphoreType.DMA,
        ],
        compiler_params=pltpu.CompilerParams(
            needs_layout_passes=False,
        ),
    )
    out = run(vt_flat, binsp, b0, bm, zeros)
    return out.transpose(1, 0, 2)


# speedup vs baseline: 1.0055x; 1.0055x over previous
import functools

import jax
import jax.numpy as jnp
from jax import lax
from jax.experimental import pallas as pl
from jax.experimental.pallas import tpu as pltpu
from jax.experimental.pallas import tpu_sc as plsc

_NW = 32
_RC = 128


def _sc_body(vals_hbm, binsp_hbm, b0_hbm, bm_hbm, zeros_hbm, out_hbm,
             vals_v, bins_v, b0_v, bm_v,
             rowbuf0, rowbuf1, colbuf0, colbuf1, sem0, sem1,
             *, rows_per_w, r0, nbins):
    nchunks = rows_per_w // _RC
    wid = lax.axis_index("s") * 2 + lax.axis_index("c")
    base = wid * rows_per_w

    pltpu.sync_copy(vals_hbm.at[pl.ds(base, rows_per_w)], vals_v)
    pltpu.sync_copy(binsp_hbm, bins_v)
    pltpu.sync_copy(b0_hbm, b0_v)
    pltpu.sync_copy(bm_hbm, bm_v)
    pltpu.sync_copy(zeros_hbm, rowbuf0)
    pltpu.sync_copy(zeros_hbm, rowbuf1)

    zf = jnp.zeros((16,), jnp.float32)
    zi = jnp.zeros((16,), jnp.int32)
    for t in range(_RC // 16):
        colbuf0[pl.ds(t * 16, 16)] = zi
        colbuf1[pl.ds(t * 16, 16)] = zi

    b0 = b0_v[pl.ds(0, 16)]
    bm = bm_v[pl.ds(0, 16)]
    lane = lax.iota(jnp.int32, 16)

    def do_chunk(c, rbuf, cbuf, sem):
        @pl.when(c >= 2)
        def _wait_prior():
            pltpu.make_async_copy(
                rbuf, out_hbm.at[0, pl.ds(0, _RC)], sem).wait()

        for t in range(_RC // 16):
            rr = t * 16 + lane
            stale = cbuf[pl.ds(t * 16, 16)]
            plsc.store_scatter(rbuf, [rr, stale], zf)
            plsc.store_scatter(rbuf, [rr, stale + 1], zf)

            v = vals_v[pl.ds(c * _RC + t * 16, 16)]
            v = jnp.minimum(jnp.maximum(v, b0), bm)
            idx = zi
            for step in (128, 64, 32, 16, 8, 4, 2, 1):
                cand = idx + step
                g = plsc.load_gather(bins_v, [cand - 1])
                idx = jnp.where(g < v, cand, idx)
            left = jnp.maximum(idx - 1, 0)
            lv = plsc.load_gather(bins_v, [left])
            rv = plsc.load_gather(bins_v, [left + 1])
            rw = (v - lv) / (rv - lv)
            lw = 1.0 - rw
            plsc.store_scatter(rbuf, [rr, left], lw)
            plsc.store_scatter(rbuf, [rr, left + 1], rw)
            cbuf[pl.ds(t * 16, 16)] = left

        g0 = base + c * _RC
        p = g0 // r0
        q = g0 - p * r0
        pltpu.make_async_copy(
            rbuf, out_hbm.at[p, pl.ds(q, _RC)], sem).start()

    def chunk_pair(pp, _):
        do_chunk(2 * pp, rowbuf0, colbuf0, sem0)
        do_chunk(2 * pp + 1, rowbuf1, colbuf1, sem1)
        return 0

    lax.fori_loop(0, nchunks // 2, chunk_pair, 0)
    pltpu.make_async_copy(rowbuf0, out_hbm.at[0, pl.ds(0, _RC)], sem0).wait()
    pltpu.make_async_copy(rowbuf1, out_hbm.at[0, pl.ds(0, _RC)], sem1).wait()


def kernel(values, bin_values):
    r0, r1 = values.shape
    nbins = bin_values.shape[0]
    n = r0 * r1
    rows_per_w = n // _NW
    vt_flat = values.T.reshape(n)
    binsp = jnp.concatenate([bin_values, bin_values[-1:]])
    b0 = jnp.broadcast_to(bin_values[0], (16,))
    bm = jnp.broadcast_to(bin_values[-1], (16,))
    zeros = jnp.zeros((_RC, nbins), jnp.float32)

    mesh = plsc.VectorSubcoreMesh(core_axis_name="c", subcore_axis_name="s")
    run = pl.kernel(
        functools.partial(_sc_body, rows_per_w=rows_per_w, r0=r0,
                          nbins=nbins),
        out_type=jax.ShapeDtypeStruct((r1, r0, nbins), jnp.float32),
        mesh=mesh,
        scratch_types=[
            pltpu.VMEM((rows_per_w,), jnp.float32),
            pltpu.VMEM((nbins + 1,), jnp.float32),
            pltpu.VMEM((16,), jnp.float32),
            pltpu.VMEM((16,), jnp.float32),
            pltpu.VMEM((_RC, nbins), jnp.float32),
            pltpu.VMEM((_RC, nbins), jnp.float32),
            pltpu.VMEM((_RC,), jnp.int32),
            pltpu.VMEM((_RC,), jnp.int32),
            pltpu.SemaphoreType.DMA,
            pltpu.SemaphoreType.DMA,
        ],
        compiler_params=pltpu.CompilerParams(
            needs_layout_passes=False,
        ),
    )
    out = run(vt_flat, binsp, b0, bm, zeros)
    return out.transpose(1, 0, 2)
